# baseline (speedup 1.0000x reference)
import functools

import jax
import jax.numpy as jnp
from jax import lax
from jax.experimental import pallas as pl
from jax.experimental.pallas import tpu as pltpu
from jax.experimental.pallas import tpu_sc as plsc

N = 10000
E = 320000
D_IN = 128
H = 64
PH = 128

NC = 2
NS = 16
NW = NC * NS
CH = 128
NCH = 80
NBUF = 4
NSLOT = 2 * NBUF
E_PER_W = NCH * CH
E_PAD = NW * E_PER_W
N_PAD = 10240
ROWS_PER_SUB = N_PAD // NS

f32 = jnp.float32


_sc_mesh = plsc.VectorSubcoreMesh(core_axis_name="c", subcore_axis_name="s")


@functools.partial(
    pl.kernel,
    out_type=jax.ShapeDtypeStruct((NC, N_PAD, H), f32),
    mesh=_sc_mesh,
    compiler_params=pltpu.CompilerParams(use_tc_tiling_on_sc=False),
    scratch_types=[
        pltpu.VMEM((NCH, CH), jnp.int32),
        pltpu.VMEM((NCH, CH), jnp.int32),
        [pltpu.VMEM((CH, H), f32) for _ in range(NSLOT)],
        pltpu.VMEM_SHARED((N_PAD, H), f32),
        [pltpu.SemaphoreType.DMA for _ in range(NSLOT)],
        [pltpu.SemaphoreType.DMA for _ in range(NSLOT)],
    ],
)
def _sc_scatter(t_hbm, src_hbm, dst_hbm, zeros_hbm, out_hbm,
                src_v, dst_v, rows, acc, gsem, ssem):
    cid = lax.axis_index("c")
    sid = lax.axis_index("s")
    wid = cid * NS + sid

    pltpu.sync_copy(zeros_hbm.at[pl.ds(sid * ROWS_PER_SUB, ROWS_PER_SUB)],
                    acc.at[pl.ds(sid * ROWS_PER_SUB, ROWS_PER_SUB)])
    pltpu.sync_copy(src_hbm.at[wid], src_v)
    pltpu.sync_copy(dst_hbm.at[wid], dst_v)
    plsc.subcore_barrier()

    for b in range(NBUF):
        pltpu.async_copy(t_hbm.at[src_v.at[b]], rows[b], gsem[b])

    def outer(step, carry):
        j0 = step * NSLOT
        for u in range(NSLOT):
            j = j0 + u
            s = u
            sk = (u + NBUF) % NSLOT
            pltpu.make_async_copy(t_hbm.at[src_v.at[j]], rows[s],
                                  gsem[s]).wait()
            pltpu.async_copy(rows[s], acc.at[dst_v.at[j]], ssem[s], add=True)
            k = j + NBUF

            @pl.when(j >= NBUF)
            def _wait_prev_scatter():
                pltpu.make_async_copy(rows[sk], acc.at[dst_v.at[j - NBUF]],
                                      ssem[sk]).wait()

            @pl.when(k < NCH)
            def _issue_next_gather():
                pltpu.async_copy(t_hbm.at[src_v.at[k]], rows[sk], gsem[sk])
        return carry

    lax.fori_loop(0, NCH // NSLOT, outer, 0, unroll=False)

    for i in range(NBUF):
        j = NCH - NBUF + i
        s = j % NSLOT
        pltpu.make_async_copy(rows[s], acc.at[dst_v.at[j]], ssem[s]).wait()

    plsc.subcore_barrier()
    pltpu.sync_copy(acc.at[pl.ds(sid * ROWS_PER_SUB, ROWS_PER_SUB)],
                    out_hbm.at[cid, pl.ds(sid * ROWS_PER_SUB, ROWS_PER_SUB)])


def _dense1_body(x_ref, wg_ref, wr_ref, br_ref, t_ref, r_ref):
    xv = x_ref[...]
    t_ref[...] = jnp.dot(xv, wg_ref[...], preferred_element_type=f32)
    r_ref[...] = jnp.maximum(
        jnp.dot(xv, wr_ref[...], preferred_element_type=f32) + br_ref[...], 0.0)


_dense1 = pl.pallas_call(
    _dense1_body,
    out_shape=[jax.ShapeDtypeStruct((N_PAD, H), f32),
               jax.ShapeDtypeStruct((N_PAD, H), f32)],
)


def _dense2_body(agg_ref, r1_ref, bg_ref, g_ref, be_ref, wg2_ref, wr2_ref,
                 br2_ref, t2_ref, r2_ref):
    agg = agg_ref[0] + agg_ref[1]
    h1 = (g_ref[...] * (jnp.maximum(agg + bg_ref[...], 0.0) + r1_ref[...])
          + be_ref[...])
    t2_ref[...] = jnp.dot(h1, wg2_ref[...], preferred_element_type=f32)
    r2_ref[...] = jnp.maximum(
        jnp.dot(h1, wr2_ref[...], preferred_element_type=f32) + br2_ref[...],
        0.0)


_dense2 = pl.pallas_call(
    _dense2_body,
    out_shape=[jax.ShapeDtypeStruct((N_PAD, H), f32),
               jax.ShapeDtypeStruct((N_PAD, H), f32)],
)


def _head_body(agg_ref, r2_ref, bg_ref, g_ref, be_ref, waw_ref, baw_ref,
               wp1_ref, bp1_ref, gp_ref, bep_ref, wp2_ref, bp2_ref,
               pred_ref, gf_ref):
    agg = agg_ref[0, :N] + agg_ref[1, :N]
    h2 = (g_ref[...] * (jnp.maximum(agg + bg_ref[...], 0.0) + r2_ref[:N])
          + be_ref[...])
    logit = jnp.sum(h2 * waw_ref[...], axis=1, keepdims=True) + baw_ref[...]
    w = jax.nn.sigmoid(logit)
    h_sum = jnp.sum(h2 * w, axis=0, keepdims=True)
    h_max = jnp.max(h2, axis=0, keepdims=True)
    gf = jnp.concatenate([h_sum, h_max], axis=1)
    z = jnp.maximum(jnp.dot(gf, wp1_ref[...], preferred_element_type=f32)
                    + bp1_ref[...], 0.0)
    z = gp_ref[...] * z + bep_ref[...]
    pred = jnp.sum(z * wp2_ref[...], axis=1, keepdims=True) + bp2_ref[...]
    pred_ref[...] = jnp.broadcast_to(pred, (1, PH))
    gf_ref[...] = gf


_head = pl.pallas_call(
    _head_body,
    out_shape=[jax.ShapeDtypeStruct((1, PH), f32),
               jax.ShapeDtypeStruct((1, 2 * H), f32)],
)


def kernel(x, edge_index, W_gc1, b_gc1, W_res1, b_res1, gamma1, beta1,
           W_gc2, b_gc2, W_res2, b_res2, gamma2, beta2, W_aw, b_aw,
           W_p1, b_p1, gamma_p, beta_p, W_p2, b_p2):
    src = edge_index[0]
    dst = edge_index[1]
    pad = E_PAD - E
    src_p = jnp.concatenate(
        [src, jnp.full((pad,), N, jnp.int32)]).reshape(NW, NCH, CH)
    dst_p = jnp.concatenate(
        [dst, jnp.full((pad,), N, jnp.int32)]).reshape(NW, NCH, CH)
    x_pad = jnp.pad(x, ((0, N_PAD - N), (0, 0)))
    zeros = jnp.zeros((N_PAD, H), f32)

    r = lambda v: v.reshape(1, -1)

    t1, r1 = _dense1(x_pad, W_gc1, W_res1, r(b_res1))
    agg1 = _sc_scatter(t1, src_p, dst_p, zeros)
    t2, r2 = _dense2(agg1, r1, r(b_gc1), r(gamma1), r(beta1),
                     W_gc2, W_res2, r(b_res2))
    agg2 = _sc_scatter(t2, src_p, dst_p, zeros)
    pred_buf, gf = _head(agg2, r2, r(b_gc2), r(gamma2), r(beta2),
                         r(W_aw), r(b_aw), W_p1, r(b_p1), r(gamma_p),
                         r(beta_p), r(W_p2), r(b_p2))
    return (pred_buf[:, :1], gf)


# speedup vs baseline: 1.8138x; 1.8138x over previous
import functools

import jax
import jax.numpy as jnp
from jax import lax
from jax.experimental import pallas as pl
from jax.experimental.pallas import tpu as pltpu
from jax.experimental.pallas import tpu_sc as plsc

N = 10000
E = 320000
D_IN = 128
H = 64
PH = 128

NC = 2
NS = 16
NW = NC * NS
CH = 128
NCH = 80
NBUF = 4
NSLOT = 2 * NBUF
E_PER_W = NCH * CH
E_PAD = NW * E_PER_W
N_PAD = 10240
ROWS_PER_SUB = N_PAD // NS

f32 = jnp.float32


_sc_mesh = plsc.VectorSubcoreMesh(core_axis_name="c", subcore_axis_name="s")


@functools.partial(
    pl.kernel,
    out_type=jax.ShapeDtypeStruct((NC, N_PAD, H), f32),
    mesh=_sc_mesh,
    compiler_params=pltpu.CompilerParams(use_tc_tiling_on_sc=False),
    scratch_types=[
        pltpu.VMEM((NCH, CH), jnp.int32),
        pltpu.VMEM((NCH, CH), jnp.int32),
        pltpu.VMEM((CH, H), f32),
        pltpu.VMEM_SHARED((N_PAD, H), f32),
        pltpu.VMEM_SHARED((N_PAD, H), f32),
        pltpu.SemaphoreType.DMA,
    ],
)
def _sc_scatter(t_hbm, src_hbm, dst_hbm, zeros_hbm, out_hbm,
                src_v, dst_v, rows, acc, tstage, gsem):
    cid = lax.axis_index("c")
    sid = lax.axis_index("s")
    wid = cid * NS + sid

    pltpu.sync_copy(t_hbm.at[pl.ds(sid * ROWS_PER_SUB, ROWS_PER_SUB)],
                    tstage.at[pl.ds(sid * ROWS_PER_SUB, ROWS_PER_SUB)])
    pltpu.sync_copy(zeros_hbm.at[pl.ds(sid * ROWS_PER_SUB, ROWS_PER_SUB)],
                    acc.at[pl.ds(sid * ROWS_PER_SUB, ROWS_PER_SUB)])
    pltpu.sync_copy(src_hbm.at[wid], src_v)
    pltpu.sync_copy(dst_hbm.at[wid], dst_v)
    plsc.subcore_barrier()

    def body1(j, _):
        pltpu.async_copy(tstage.at[src_v.at[j]], rows, gsem).wait()
        pltpu.sync_copy(rows, acc.at[dst_v.at[j]], add=True)
        return _

    lax.fori_loop(0, NCH, body1, 0, unroll=False)

    plsc.subcore_barrier()
    pltpu.sync_copy(acc.at[pl.ds(sid * ROWS_PER_SUB, ROWS_PER_SUB)],
                    out_hbm.at[cid, pl.ds(sid * ROWS_PER_SUB, ROWS_PER_SUB)])


def _dense1_body(x_ref, wg_ref, wr_ref, br_ref, t_ref, r_ref):
    xv = x_ref[...]
    t_ref[...] = jnp.dot(xv, wg_ref[...], preferred_element_type=f32)
    r_ref[...] = jnp.maximum(
        jnp.dot(xv, wr_ref[...], preferred_element_type=f32) + br_ref[...], 0.0)


_dense1 = pl.pallas_call(
    _dense1_body,
    out_shape=[jax.ShapeDtypeStruct((N_PAD, H), f32),
               jax.ShapeDtypeStruct((N_PAD, H), f32)],
)


def _dense2_body(agg_ref, r1_ref, bg_ref, g_ref, be_ref, wg2_ref, wr2_ref,
                 br2_ref, t2_ref, r2_ref):
    agg = agg_ref[0] + agg_ref[1]
    h1 = (g_ref[...] * (jnp.maximum(agg + bg_ref[...], 0.0) + r1_ref[...])
          + be_ref[...])
    t2_ref[...] = jnp.dot(h1, wg2_ref[...], preferred_element_type=f32)
    r2_ref[...] = jnp.maximum(
        jnp.dot(h1, wr2_ref[...], preferred_element_type=f32) + br2_ref[...],
        0.0)


_dense2 = pl.pallas_call(
    _dense2_body,
    out_shape=[jax.ShapeDtypeStruct((N_PAD, H), f32),
               jax.ShapeDtypeStruct((N_PAD, H), f32)],
)


def _head_body(agg_ref, r2_ref, bg_ref, g_ref, be_ref, waw_ref, baw_ref,
               wp1_ref, bp1_ref, gp_ref, bep_ref, wp2_ref, bp2_ref,
               pred_ref, gf_ref):
    agg = agg_ref[0, :N] + agg_ref[1, :N]
    h2 = (g_ref[...] * (jnp.maximum(agg + bg_ref[...], 0.0) + r2_ref[:N])
          + be_ref[...])
    logit = jnp.sum(h2 * waw_ref[...], axis=1, keepdims=True) + baw_ref[...]
    w = jax.nn.sigmoid(logit)
    h_sum = jnp.sum(h2 * w, axis=0, keepdims=True)
    h_max = jnp.max(h2, axis=0, keepdims=True)
    gf = jnp.concatenate([h_sum, h_max], axis=1)
    z = jnp.maximum(jnp.dot(gf, wp1_ref[...], preferred_element_type=f32)
                    + bp1_ref[...], 0.0)
    z = gp_ref[...] * z + bep_ref[...]
    pred = jnp.sum(z * wp2_ref[...], axis=1, keepdims=True) + bp2_ref[...]
    pred_ref[...] = jnp.broadcast_to(pred, (1, PH))
    gf_ref[...] = gf


_head = pl.pallas_call(
    _head_body,
    out_shape=[jax.ShapeDtypeStruct((1, PH), f32),
               jax.ShapeDtypeStruct((1, 2 * H), f32)],
)


def kernel(x, edge_index, W_gc1, b_gc1, W_res1, b_res1, gamma1, beta1,
           W_gc2, b_gc2, W_res2, b_res2, gamma2, beta2, W_aw, b_aw,
           W_p1, b_p1, gamma_p, beta_p, W_p2, b_p2):
    src = edge_index[0]
    dst = edge_index[1]
    pad = E_PAD - E
    src_p = jnp.concatenate(
        [src, jnp.full((pad,), N, jnp.int32)]).reshape(NW, NCH, CH)
    dst_p = jnp.concatenate(
        [dst, jnp.full((pad,), N, jnp.int32)]).reshape(NW, NCH, CH)
    x_pad = jnp.pad(x, ((0, N_PAD - N), (0, 0)))
    zeros = jnp.zeros((N_PAD, H), f32)

    r = lambda v: v.reshape(1, -1)

    t1, r1 = _dense1(x_pad, W_gc1, W_res1, r(b_res1))
    agg1 = _sc_scatter(t1, src_p, dst_p, zeros)
    t2, r2 = _dense2(agg1, r1, r(b_gc1), r(gamma1), r(beta1),
                     W_gc2, W_res2, r(b_res2))
    agg2 = _sc_scatter(t2, src_p, dst_p, zeros)
    pred_buf, gf = _head(agg2, r2, r(b_gc2), r(gamma2), r(beta2),
                         r(W_aw), r(b_aw), W_p1, r(b_p1), r(gamma_p),
                         r(beta_p), r(W_p2), r(b_p2))
    return (pred_buf[:, :1], gf)


# speedup vs baseline: 2.2676x; 1.2502x over previous
import functools

import jax
import jax.numpy as jnp
from jax import lax
from jax.experimental import pallas as pl
from jax.experimental.pallas import tpu as pltpu
from jax.experimental.pallas import tpu_sc as plsc

N = 10000
E = 320000
D_IN = 128
H = 64
PH = 128

NC = 2
NS = 16
NW = NC * NS
CH = 128
NCH = 80
NBUF = 4
NSLOT = 2 * NBUF
E_PER_W = NCH * CH
E_PAD = NW * E_PER_W
N_PAD = 10240
ROWS_PER_SUB = N_PAD // NS

f32 = jnp.float32


_sc_mesh = plsc.VectorSubcoreMesh(core_axis_name="c", subcore_axis_name="s")


@functools.partial(
    pl.kernel,
    out_type=jax.ShapeDtypeStruct((NC, N_PAD, H), f32),
    mesh=_sc_mesh,
    compiler_params=pltpu.CompilerParams(use_tc_tiling_on_sc=False),
    scratch_types=[
        pltpu.VMEM((NCH, CH), jnp.int32),
        pltpu.VMEM((NCH, CH), jnp.int32),
        [pltpu.VMEM((CH, H), f32) for _ in range(2)],
        pltpu.VMEM_SHARED((N_PAD, H), f32),
        pltpu.VMEM_SHARED((N_PAD, H), f32),
        [pltpu.SemaphoreType.DMA for _ in range(2)],
    ],
)
def _sc_scatter(t_hbm, src_hbm, dst_hbm, zeros_hbm, out_hbm,
                src_v, dst_v, rows, acc, tstage, gsem):
    cid = lax.axis_index("c")
    sid = lax.axis_index("s")
    wid = cid * NS + sid

    pltpu.sync_copy(t_hbm.at[pl.ds(sid * ROWS_PER_SUB, ROWS_PER_SUB)],
                    tstage.at[pl.ds(sid * ROWS_PER_SUB, ROWS_PER_SUB)])
    pltpu.sync_copy(zeros_hbm.at[pl.ds(sid * ROWS_PER_SUB, ROWS_PER_SUB)],
                    acc.at[pl.ds(sid * ROWS_PER_SUB, ROWS_PER_SUB)])
    pltpu.sync_copy(src_hbm.at[wid], src_v)
    pltpu.sync_copy(dst_hbm.at[wid], dst_v)
    plsc.subcore_barrier()

    pltpu.async_copy(tstage.at[src_v.at[0]], rows[0], gsem[0])

    def body1(step, _):
        for u in range(2):
            j = step * 2 + u
            k = j + 1

            @pl.when(k < NCH)
            def _prefetch():
                pltpu.async_copy(tstage.at[src_v.at[k]], rows[1 - u],
                                 gsem[1 - u])

            pltpu.make_async_copy(tstage.at[src_v.at[j]], rows[u],
                                  gsem[u]).wait()
            pltpu.sync_copy(rows[u], acc.at[dst_v.at[j]], add=True)
        return _

    lax.fori_loop(0, NCH // 2, body1, 0, unroll=False)

    plsc.subcore_barrier()
    pltpu.sync_copy(acc.at[pl.ds(sid * ROWS_PER_SUB, ROWS_PER_SUB)],
                    out_hbm.at[cid, pl.ds(sid * ROWS_PER_SUB, ROWS_PER_SUB)])


def _dense1_body(x_ref, wg_ref, wr_ref, br_ref, t_ref, r_ref):
    xv = x_ref[...]
    t_ref[...] = jnp.dot(xv, wg_ref[...], preferred_element_type=f32)
    r_ref[...] = jnp.maximum(
        jnp.dot(xv, wr_ref[...], preferred_element_type=f32) + br_ref[...], 0.0)


_dense1 = pl.pallas_call(
    _dense1_body,
    out_shape=[jax.ShapeDtypeStruct((N_PAD, H), f32),
               jax.ShapeDtypeStruct((N_PAD, H), f32)],
)


def _dense2_body(agg_ref, r1_ref, bg_ref, g_ref, be_ref, wg2_ref, wr2_ref,
                 br2_ref, t2_ref, r2_ref):
    agg = agg_ref[0] + agg_ref[1]
    h1 = (g_ref[...] * (jnp.maximum(agg + bg_ref[...], 0.0) + r1_ref[...])
          + be_ref[...])
    t2_ref[...] = jnp.dot(h1, wg2_ref[...], preferred_element_type=f32)
    r2_ref[...] = jnp.maximum(
        jnp.dot(h1, wr2_ref[...], preferred_element_type=f32) + br2_ref[...],
        0.0)


_dense2 = pl.pallas_call(
    _dense2_body,
    out_shape=[jax.ShapeDtypeStruct((N_PAD, H), f32),
               jax.ShapeDtypeStruct((N_PAD, H), f32)],
)


def _head_body(agg_ref, r2_ref, bg_ref, g_ref, be_ref, waw_ref, baw_ref,
               wp1_ref, bp1_ref, gp_ref, bep_ref, wp2_ref, bp2_ref,
               pred_ref, gf_ref):
    agg = agg_ref[0, :N] + agg_ref[1, :N]
    h2 = (g_ref[...] * (jnp.maximum(agg + bg_ref[...], 0.0) + r2_ref[:N])
          + be_ref[...])
    logit = jnp.sum(h2 * waw_ref[...], axis=1, keepdims=True) + baw_ref[...]
    w = jax.nn.sigmoid(logit)
    h_sum = jnp.sum(h2 * w, axis=0, keepdims=True)
    h_max = jnp.max(h2, axis=0, keepdims=True)
    gf = jnp.concatenate([h_sum, h_max], axis=1)
    z = jnp.maximum(jnp.dot(gf, wp1_ref[...], preferred_element_type=f32)
                    + bp1_ref[...], 0.0)
    z = gp_ref[...] * z + bep_ref[...]
    pred = jnp.sum(z * wp2_ref[...], axis=1, keepdims=True) + bp2_ref[...]
    pred_ref[...] = jnp.broadcast_to(pred, (1, PH))
    gf_ref[...] = gf


_head = pl.pallas_call(
    _head_body,
    out_shape=[jax.ShapeDtypeStruct((1, PH), f32),
               jax.ShapeDtypeStruct((1, 2 * H), f32)],
)


def kernel(x, edge_index, W_gc1, b_gc1, W_res1, b_res1, gamma1, beta1,
           W_gc2, b_gc2, W_res2, b_res2, gamma2, beta2, W_aw, b_aw,
           W_p1, b_p1, gamma_p, beta_p, W_p2, b_p2):
    src = edge_index[0]
    dst = edge_index[1]
    pad = E_PAD - E
    src_p = jnp.concatenate(
        [src, jnp.full((pad,), N, jnp.int32)]).reshape(NW, NCH, CH)
    dst_p = jnp.concatenate(
        [dst, jnp.full((pad,), N, jnp.int32)]).reshape(NW, NCH, CH)
    x_pad = jnp.pad(x, ((0, N_PAD - N), (0, 0)))
    zeros = jnp.zeros((N_PAD, H), f32)

    r = lambda v: v.reshape(1, -1)

    t1, r1 = _dense1(x_pad, W_gc1, W_res1, r(b_res1))
    agg1 = _sc_scatter(t1, src_p, dst_p, zeros)
    t2, r2 = _dense2(agg1, r1, r(b_gc1), r(gamma1), r(beta1),
                     W_gc2, W_res2, r(b_res2))
    agg2 = _sc_scatter(t2, src_p, dst_p, zeros)
    pred_buf, gf = _head(agg2, r2, r(b_gc2), r(gamma2), r(beta2),
                         r(W_aw), r(b_aw), W_p1, r(b_p1), r(gamma_p),
                         r(beta_p), r(W_p2), r(b_p2))
    return (pred_buf[:, :1], gf)


# speedup vs baseline: 2.2695x; 1.0008x over previous
import functools

import jax
import jax.numpy as jnp
from jax import lax
from jax.experimental import pallas as pl
from jax.experimental.pallas import tpu as pltpu
from jax.experimental.pallas import tpu_sc as plsc

N = 10000
E = 320000
D_IN = 128
H = 64
PH = 128

NC = 2
NS = 16
NW = NC * NS
CH = 128
NCH = 80
NBUF = 4
NSLOT = 2 * NBUF
E_PER_W = NCH * CH
E_PAD = NW * E_PER_W
N_PAD = 10240
ROWS_PER_SUB = N_PAD // NS

f32 = jnp.float32


_sc_mesh = plsc.VectorSubcoreMesh(core_axis_name="c", subcore_axis_name="s")


@functools.partial(
    pl.kernel,
    out_type=jax.ShapeDtypeStruct((NC, N_PAD, H), f32),
    mesh=_sc_mesh,
    compiler_params=pltpu.CompilerParams(use_tc_tiling_on_sc=False),
    scratch_types=[
        pltpu.VMEM((NCH, CH), jnp.int32),
        pltpu.VMEM((NCH, CH), jnp.int32),
        [pltpu.VMEM((CH, H), f32) for _ in range(2)],
        pltpu.VMEM_SHARED((N_PAD, H), f32),
        pltpu.VMEM_SHARED((N_PAD, H), f32),
        [pltpu.SemaphoreType.DMA for _ in range(2)],
    ],
)
def _sc_scatter(t_hbm, src_hbm, dst_hbm, zeros_hbm, out_hbm,
                src_v, dst_v, rows, acc, tstage, gsem):
    cid = lax.axis_index("c")
    sid = lax.axis_index("s")
    wid = cid * NS + sid

    pltpu.sync_copy(t_hbm.at[pl.ds(sid * ROWS_PER_SUB, ROWS_PER_SUB)],
                    tstage.at[pl.ds(sid * ROWS_PER_SUB, ROWS_PER_SUB)])
    pltpu.sync_copy(zeros_hbm.at[pl.ds(sid * ROWS_PER_SUB, ROWS_PER_SUB)],
                    acc.at[pl.ds(sid * ROWS_PER_SUB, ROWS_PER_SUB)])
    pltpu.sync_copy(src_hbm.at[wid], src_v)
    pltpu.sync_copy(dst_hbm.at[wid], dst_v)
    plsc.subcore_barrier()

    pltpu.async_copy(tstage.at[src_v.at[0]], rows[0], gsem[0])

    def body1(step, _):
        for u in range(2):
            j = step * 2 + u
            k = j + 1

            @pl.when(k < NCH)
            def _prefetch():
                pltpu.async_copy(tstage.at[src_v.at[k]], rows[1 - u],
                                 gsem[1 - u])

            pltpu.make_async_copy(tstage.at[src_v.at[j]], rows[u],
                                  gsem[u]).wait()
            pltpu.sync_copy(rows[u], acc.at[dst_v.at[j]], add=True)
        return _

    lax.fori_loop(0, NCH // 2, body1, 0, unroll=False)

    plsc.subcore_barrier()
    pltpu.sync_copy(acc.at[pl.ds(sid * ROWS_PER_SUB, ROWS_PER_SUB)],
                    out_hbm.at[cid, pl.ds(sid * ROWS_PER_SUB, ROWS_PER_SUB)])


def _dense1_body(x_ref, wg_ref, wr_ref, br_ref, t_ref, r_ref):
    xv = x_ref[...]
    t_ref[...] = jnp.dot(xv, wg_ref[...], preferred_element_type=f32)
    r_ref[...] = jnp.maximum(
        jnp.dot(xv, wr_ref[...], preferred_element_type=f32) + br_ref[...], 0.0)


_dense1 = pl.pallas_call(
    _dense1_body,
    out_shape=[jax.ShapeDtypeStruct((N_PAD, H), f32),
               jax.ShapeDtypeStruct((N_PAD, H), f32)],
)


def _dense2_body(agg_ref, r1_ref, bg_ref, g_ref, be_ref, wg2_ref, wr2_ref,
                 br2_ref, t2_ref, r2_ref):
    agg = agg_ref[0] + agg_ref[1]
    h1 = (g_ref[...] * (jnp.maximum(agg + bg_ref[...], 0.0) + r1_ref[...])
          + be_ref[...])
    t2_ref[...] = jnp.dot(h1, wg2_ref[...], preferred_element_type=f32)
    r2_ref[...] = jnp.maximum(
        jnp.dot(h1, wr2_ref[...], preferred_element_type=f32) + br2_ref[...],
        0.0)


_dense2 = pl.pallas_call(
    _dense2_body,
    out_shape=[jax.ShapeDtypeStruct((N_PAD, H), f32),
               jax.ShapeDtypeStruct((N_PAD, H), f32)],
)


def _head_body(agg_ref, r2_ref, bg_ref, g_ref, be_ref, waw_ref, baw_ref,
               wp1_ref, bp1_ref, gp_ref, bep_ref, wp2_ref, bp2_ref,
               pred_ref, gf_ref):
    agg = agg_ref[0, :N] + agg_ref[1, :N]
    h2 = (g_ref[...] * (jnp.maximum(agg + bg_ref[...], 0.0) + r2_ref[:N])
          + be_ref[...])
    logits = jnp.dot(h2, waw_ref[...], preferred_element_type=f32)
    w = jax.nn.sigmoid(logits[:, :1] + baw_ref[...])
    h_sum = jnp.sum(h2 * w, axis=0, keepdims=True)
    h_max = jnp.max(h2, axis=0, keepdims=True)
    gf = jnp.concatenate([h_sum, h_max], axis=1)
    z = bp1_ref[...]
    for k in range(2 * H):
        z = z + gf[0, k] * wp1_ref[k:k + 1, :]
    z = jnp.maximum(z, 0.0)
    z = gp_ref[...] * z + bep_ref[...]
    pred = jnp.sum(z * wp2_ref[...], axis=1, keepdims=True) + bp2_ref[...]
    pred_ref[...] = jnp.broadcast_to(pred, (1, PH))
    gf_ref[...] = gf


_head = pl.pallas_call(
    _head_body,
    out_shape=[jax.ShapeDtypeStruct((1, PH), f32),
               jax.ShapeDtypeStruct((1, 2 * H), f32)],
)


def kernel(x, edge_index, W_gc1, b_gc1, W_res1, b_res1, gamma1, beta1,
           W_gc2, b_gc2, W_res2, b_res2, gamma2, beta2, W_aw, b_aw,
           W_p1, b_p1, gamma_p, beta_p, W_p2, b_p2):
    src = edge_index[0]
    dst = edge_index[1]
    pad = E_PAD - E
    src_p = jnp.concatenate(
        [src, jnp.full((pad,), N, jnp.int32)]).reshape(NW, NCH, CH)
    dst_p = jnp.concatenate(
        [dst, jnp.full((pad,), N, jnp.int32)]).reshape(NW, NCH, CH)
    x_pad = jnp.pad(x, ((0, N_PAD - N), (0, 0)))
    zeros = jnp.zeros((N_PAD, H), f32)

    r = lambda v: v.reshape(1, -1)

    t1, r1 = _dense1(x_pad, W_gc1, W_res1, r(b_res1))
    agg1 = _sc_scatter(t1, src_p, dst_p, zeros)
    t2, r2 = _dense2(agg1, r1, r(b_gc1), r(gamma1), r(beta1),
                     W_gc2, W_res2, r(b_res2))
    agg2 = _sc_scatter(t2, src_p, dst_p, zeros)
    waw_pad = jnp.pad(W_aw, ((0, 0), (0, 127)))
    pred_buf, gf = _head(agg2, r2, r(b_gc2), r(gamma2), r(beta2),
                         waw_pad, r(b_aw), W_p1, r(b_p1), r(gamma_p),
                         r(beta_p), r(W_p2), r(b_p2))
    return (pred_buf[:, :1], gf)


# speedup vs baseline: 2.4684x; 1.0876x over previous
import functools

import jax
import jax.numpy as jnp
from jax import lax
from jax.experimental import pallas as pl
from jax.experimental.pallas import tpu as pltpu
from jax.experimental.pallas import tpu_sc as plsc

N = 10000
E = 320000
D_IN = 128
H = 64
PH = 128

NC = 2
NS = 16
NW = NC * NS
CH = 96
NCH = 108
NBUF = 2
NSLOT = 2 * NBUF
E_PER_W = NCH * CH
E_PAD = NW * E_PER_W
N_PAD = 10240
ROWS_PER_SUB = N_PAD // NS

f32 = jnp.float32


_sc_mesh = plsc.VectorSubcoreMesh(core_axis_name="c", subcore_axis_name="s")


@functools.partial(
    pl.kernel,
    out_type=jax.ShapeDtypeStruct((NC, N_PAD, H), f32),
    mesh=_sc_mesh,
    compiler_params=pltpu.CompilerParams(use_tc_tiling_on_sc=False),
    scratch_types=[
        pltpu.VMEM((NCH, CH), jnp.int32),
        pltpu.VMEM((NCH, CH), jnp.int32),
        [pltpu.VMEM((CH, H), f32) for _ in range(NSLOT)],
        pltpu.VMEM_SHARED((N_PAD, H), f32),
        pltpu.VMEM_SHARED((N_PAD, H), f32),
        [pltpu.SemaphoreType.DMA for _ in range(NSLOT)],
        [pltpu.SemaphoreType.DMA for _ in range(NSLOT)],
    ],
)
def _sc_scatter(t_hbm, src_hbm, dst_hbm, zeros_hbm, out_hbm,
                src_v, dst_v, rows, acc, tstage, gsem, ssem):
    cid = lax.axis_index("c")
    sid = lax.axis_index("s")
    wid = cid * NS + sid

    pltpu.sync_copy(t_hbm.at[pl.ds(sid * ROWS_PER_SUB, ROWS_PER_SUB)],
                    tstage.at[pl.ds(sid * ROWS_PER_SUB, ROWS_PER_SUB)])
    pltpu.sync_copy(zeros_hbm.at[pl.ds(sid * ROWS_PER_SUB, ROWS_PER_SUB)],
                    acc.at[pl.ds(sid * ROWS_PER_SUB, ROWS_PER_SUB)])
    pltpu.sync_copy(src_hbm.at[wid], src_v)
    pltpu.sync_copy(dst_hbm.at[wid], dst_v)
    plsc.subcore_barrier()

    for b in range(NBUF):
        pltpu.async_copy(tstage.at[src_v.at[b]], rows[b], gsem[b])

    def body1(step, _):
        j0 = step * NSLOT
        for u in range(NSLOT):
            j = j0 + u
            sk = (u + NBUF) % NSLOT
            pltpu.make_async_copy(tstage.at[src_v.at[j]], rows[u],
                                  gsem[u]).wait()
            pltpu.async_copy(rows[u], acc.at[dst_v.at[j]], ssem[u], add=True)
            k = j + NBUF

            @pl.when(j >= NBUF)
            def _wait_prev_scatter():
                pltpu.make_async_copy(rows[sk], acc.at[dst_v.at[j - NBUF]],
                                      ssem[sk]).wait()

            @pl.when(k < NCH)
            def _issue_next_gather():
                pltpu.async_copy(tstage.at[src_v.at[k]], rows[sk], gsem[sk])
        return _

    lax.fori_loop(0, NCH // NSLOT, body1, 0, unroll=False)

    for i in range(NBUF):
        j = NCH - NBUF + i
        s = j % NSLOT
        pltpu.make_async_copy(rows[s], acc.at[dst_v.at[j]], ssem[s]).wait()

    plsc.subcore_barrier()
    pltpu.sync_copy(acc.at[pl.ds(sid * ROWS_PER_SUB, ROWS_PER_SUB)],
                    out_hbm.at[cid, pl.ds(sid * ROWS_PER_SUB, ROWS_PER_SUB)])


def _dense1_body(x_ref, wg_ref, wr_ref, br_ref, t_ref, r_ref):
    xv = x_ref[...]
    t_ref[...] = jnp.dot(xv, wg_ref[...], preferred_element_type=f32)
    r_ref[...] = jnp.maximum(
        jnp.dot(xv, wr_ref[...], preferred_element_type=f32) + br_ref[...], 0.0)


_dense1 = pl.pallas_call(
    _dense1_body,
    out_shape=[jax.ShapeDtypeStruct((N_PAD, H), f32),
               jax.ShapeDtypeStruct((N_PAD, H), f32)],
)


def _dense2_body(agg_ref, r1_ref, bg_ref, g_ref, be_ref, wg2_ref, wr2_ref,
                 br2_ref, t2_ref, r2_ref):
    agg = agg_ref[0] + agg_ref[1]
    h1 = (g_ref[...] * (jnp.maximum(agg + bg_ref[...], 0.0) + r1_ref[...])
          + be_ref[...])
    t2_ref[...] = jnp.dot(h1, wg2_ref[...], preferred_element_type=f32)
    r2_ref[...] = jnp.maximum(
        jnp.dot(h1, wr2_ref[...], preferred_element_type=f32) + br2_ref[...],
        0.0)


_dense2 = pl.pallas_call(
    _dense2_body,
    out_shape=[jax.ShapeDtypeStruct((N_PAD, H), f32),
               jax.ShapeDtypeStruct((N_PAD, H), f32)],
)


def _head_body(agg_ref, r2_ref, bg_ref, g_ref, be_ref, waw_ref, baw_ref,
               wp1_ref, bp1_ref, gp_ref, bep_ref, wp2_ref, bp2_ref,
               pred_ref, gf_ref):
    agg = agg_ref[0, :N] + agg_ref[1, :N]
    h2 = (g_ref[...] * (jnp.maximum(agg + bg_ref[...], 0.0) + r2_ref[:N])
          + be_ref[...])
    logits = jnp.dot(h2, waw_ref[...], preferred_element_type=f32)
    w = jax.nn.sigmoid(logits[:, :1] + baw_ref[...])
    h_sum = jnp.sum(h2 * w, axis=0, keepdims=True)
    h_max = jnp.max(h2, axis=0, keepdims=True)
    gf = jnp.concatenate([h_sum, h_max], axis=1)
    z = bp1_ref[...]
    for k in range(2 * H):
        z = z + gf[0, k] * wp1_ref[k:k + 1, :]
    z = jnp.maximum(z, 0.0)
    z = gp_ref[...] * z + bep_ref[...]
    pred = jnp.sum(z * wp2_ref[...], axis=1, keepdims=True) + bp2_ref[...]
    pred_ref[...] = jnp.broadcast_to(pred, (1, PH))
    gf_ref[...] = gf


_head = pl.pallas_call(
    _head_body,
    out_shape=[jax.ShapeDtypeStruct((1, PH), f32),
               jax.ShapeDtypeStruct((1, 2 * H), f32)],
)


def kernel(x, edge_index, W_gc1, b_gc1, W_res1, b_res1, gamma1, beta1,
           W_gc2, b_gc2, W_res2, b_res2, gamma2, beta2, W_aw, b_aw,
           W_p1, b_p1, gamma_p, beta_p, W_p2, b_p2):
    src = edge_index[0]
    dst = edge_index[1]
    pad = E_PAD - E
    src_p = jnp.concatenate(
        [src, jnp.full((pad,), N, jnp.int32)]).reshape(NW, NCH, CH)
    dst_p = jnp.concatenate(
        [dst, jnp.full((pad,), N, jnp.int32)]).reshape(NW, NCH, CH)
    x_pad = jnp.pad(x, ((0, N_PAD - N), (0, 0)))
    zeros = jnp.zeros((N_PAD, H), f32)

    r = lambda v: v.reshape(1, -1)

    t1, r1 = _dense1(x_pad, W_gc1, W_res1, r(b_res1))
    agg1 = _sc_scatter(t1, src_p, dst_p, zeros)
    t2, r2 = _dense2(agg1, r1, r(b_gc1), r(gamma1), r(beta1),
                     W_gc2, W_res2, r(b_res2))
    agg2 = _sc_scatter(t2, src_p, dst_p, zeros)
    waw_pad = jnp.pad(W_aw, ((0, 0), (0, 127)))
    pred_buf, gf = _head(agg2, r2, r(b_gc2), r(gamma2), r(beta2),
                         waw_pad, r(b_aw), W_p1, r(b_p1), r(gamma_p),
                         r(beta_p), r(W_p2), r(b_p2))
    return (pred_buf[:, :1], gf)


# speedup vs baseline: 2.4974x; 1.0118x over previous
import functools

import jax
import jax.numpy as jnp
from jax import lax
from jax.experimental import pallas as pl
from jax.experimental.pallas import tpu as pltpu
from jax.experimental.pallas import tpu_sc as plsc

N = 10000
E = 320000
D_IN = 128
H = 64
PH = 128

NC = 2
NS = 16
NW = NC * NS
CH = 96
NCH = 108
NBUF = 2
NSLOT = 2 * NBUF
E_PER_W = NCH * CH
E_PAD = NW * E_PER_W
N_PAD = 10240
ROWS_PER_SUB = N_PAD // NS

f32 = jnp.float32


_sc_mesh = plsc.VectorSubcoreMesh(core_axis_name="c", subcore_axis_name="s")


@functools.partial(
    pl.kernel,
    out_type=jax.ShapeDtypeStruct((NC, N_PAD, H), f32),
    mesh=_sc_mesh,
    compiler_params=pltpu.CompilerParams(use_tc_tiling_on_sc=False),
    scratch_types=[
        pltpu.VMEM((NCH, CH), jnp.int32),
        pltpu.VMEM((NCH, CH), jnp.int32),
        [pltpu.VMEM((CH, H), f32) for _ in range(NSLOT)],
        pltpu.VMEM_SHARED((N_PAD, H), f32),
        pltpu.VMEM_SHARED((N_PAD, H), f32),
        [pltpu.SemaphoreType.DMA for _ in range(NSLOT)],
        [pltpu.SemaphoreType.DMA for _ in range(NSLOT)],
    ],
)
def _sc_scatter(t_hbm, src_hbm, dst_hbm, zeros_hbm, out_hbm,
                src_v, dst_v, rows, acc, tstage, gsem, ssem):
    cid = lax.axis_index("c")
    sid = lax.axis_index("s")
    wid = cid * NS + sid

    pltpu.sync_copy(t_hbm.at[pl.ds(sid * ROWS_PER_SUB, ROWS_PER_SUB)],
                    tstage.at[pl.ds(sid * ROWS_PER_SUB, ROWS_PER_SUB)])
    pltpu.sync_copy(zeros_hbm.at[pl.ds(sid * ROWS_PER_SUB, ROWS_PER_SUB)],
                    acc.at[pl.ds(sid * ROWS_PER_SUB, ROWS_PER_SUB)])
    pltpu.sync_copy(src_hbm.at[wid], src_v)
    pltpu.sync_copy(dst_hbm.at[wid], dst_v)
    plsc.subcore_barrier()

    for b in range(NBUF):
        pltpu.async_copy(tstage.at[src_v.at[b]], rows[b], gsem[b])

    def body1(step, _):
        j0 = step * NSLOT
        for u in range(NSLOT):
            j = j0 + u
            sk = (u + NBUF) % NSLOT
            pltpu.make_async_copy(tstage.at[src_v.at[j]], rows[u],
                                  gsem[u]).wait()
            pltpu.async_copy(rows[u], acc.at[dst_v.at[j]], ssem[u], add=True)
            k = j + NBUF

            @pl.when(j >= NBUF)
            def _wait_prev_scatter():
                pltpu.make_async_copy(rows[sk], acc.at[dst_v.at[j - NBUF]],
                                      ssem[sk]).wait()

            @pl.when(k < NCH)
            def _issue_next_gather():
                pltpu.async_copy(tstage.at[src_v.at[k]], rows[sk], gsem[sk])
        return _

    lax.fori_loop(0, NCH // NSLOT, body1, 0, unroll=False)

    for i in range(NBUF):
        j = NCH - NBUF + i
        s = j % NSLOT
        pltpu.make_async_copy(rows[s], acc.at[dst_v.at[j]], ssem[s]).wait()

    plsc.subcore_barrier()
    pltpu.sync_copy(acc.at[pl.ds(sid * ROWS_PER_SUB, ROWS_PER_SUB)],
                    out_hbm.at[cid, pl.ds(sid * ROWS_PER_SUB, ROWS_PER_SUB)])


def _dense1_body(x_ref, wg_ref, wr_ref, br_ref, t_ref, r_ref):
    xv = x_ref[...]
    t_ref[N:, :] = jnp.zeros((N_PAD - N, H), f32)
    r_ref[N:, :] = jnp.zeros((N_PAD - N, H), f32)
    t_ref[:N, :] = jnp.dot(xv, wg_ref[...], preferred_element_type=f32)
    r_ref[:N, :] = jnp.maximum(
        jnp.dot(xv, wr_ref[...], preferred_element_type=f32) + br_ref[...], 0.0)


_dense1 = pl.pallas_call(
    _dense1_body,
    out_shape=[jax.ShapeDtypeStruct((N_PAD, H), f32),
               jax.ShapeDtypeStruct((N_PAD, H), f32)],
)


def _dense2_body(agg_ref, r1_ref, bg_ref, g_ref, be_ref, wg2_ref, wr2_ref,
                 br2_ref, t2_ref, r2_ref):
    agg = agg_ref[0] + agg_ref[1]
    h1 = (g_ref[...] * (jnp.maximum(agg + bg_ref[...], 0.0) + r1_ref[...])
          + be_ref[...])
    t2_ref[...] = jnp.dot(h1, wg2_ref[...], preferred_element_type=f32)
    r2_ref[...] = jnp.maximum(
        jnp.dot(h1, wr2_ref[...], preferred_element_type=f32) + br2_ref[...],
        0.0)


_dense2 = pl.pallas_call(
    _dense2_body,
    out_shape=[jax.ShapeDtypeStruct((N_PAD, H), f32),
               jax.ShapeDtypeStruct((N_PAD, H), f32)],
)


def _head_body(agg_ref, r2_ref, bg_ref, g_ref, be_ref, waw_ref, baw_ref,
               wp1_ref, bp1_ref, gp_ref, bep_ref, wp2_ref, bp2_ref,
               pred_ref, gf_ref):
    agg = agg_ref[0, :N] + agg_ref[1, :N]
    h2 = (g_ref[...] * (jnp.maximum(agg + bg_ref[...], 0.0) + r2_ref[:N])
          + be_ref[...])
    logits = jnp.dot(h2, waw_ref[...], preferred_element_type=f32)
    w = jax.nn.sigmoid(logits[:, :1] + baw_ref[...])
    h_sum = jnp.sum(h2 * w, axis=0, keepdims=True)
    h_max = jnp.max(h2, axis=0, keepdims=True)
    gf = jnp.concatenate([h_sum, h_max], axis=1)
    z = bp1_ref[...]
    for k in range(2 * H):
        z = z + gf[0, k] * wp1_ref[k:k + 1, :]
    z = jnp.maximum(z, 0.0)
    z = gp_ref[...] * z + bep_ref[...]
    pred = jnp.sum(z * wp2_ref[...], axis=1, keepdims=True) + bp2_ref[...]
    pred_ref[...] = jnp.broadcast_to(pred, (1, PH))
    gf_ref[...] = gf


_head = pl.pallas_call(
    _head_body,
    out_shape=[jax.ShapeDtypeStruct((1, PH), f32),
               jax.ShapeDtypeStruct((1, 2 * H), f32)],
)


def kernel(x, edge_index, W_gc1, b_gc1, W_res1, b_res1, gamma1, beta1,
           W_gc2, b_gc2, W_res2, b_res2, gamma2, beta2, W_aw, b_aw,
           W_p1, b_p1, gamma_p, beta_p, W_p2, b_p2):
    src = edge_index[0]
    dst = edge_index[1]
    pad = E_PAD - E
    src_p = jnp.concatenate(
        [src, jnp.full((pad,), N, jnp.int32)]).reshape(NW, NCH, CH)
    dst_p = jnp.concatenate(
        [dst, jnp.full((pad,), N, jnp.int32)]).reshape(NW, NCH, CH)
    zeros = jnp.zeros((N_PAD, H), f32)

    r = lambda v: v.reshape(1, -1)

    t1, r1 = _dense1(x, W_gc1, W_res1, r(b_res1))
    agg1 = _sc_scatter(t1, src_p, dst_p, zeros)
    t2, r2 = _dense2(agg1, r1, r(b_gc1), r(gamma1), r(beta1),
                     W_gc2, W_res2, r(b_res2))
    agg2 = _sc_scatter(t2, src_p, dst_p, zeros)
    waw_pad = jnp.pad(W_aw, ((0, 0), (0, 127)))
    pred_buf, gf = _head(agg2, r2, r(b_gc2), r(gamma2), r(beta2),
                         waw_pad, r(b_aw), W_p1, r(b_p1), r(gamma_p),
                         r(beta_p), r(W_p2), r(b_p2))
    return (pred_buf[:, :1], gf)
